# speedup vs baseline: 1.4197x; 1.4197x over previous
import functools

import jax
import jax.numpy as jnp
from jax import lax
from jax.experimental import pallas as pl
from jax.experimental.pallas import tpu as pltpu
from jax.experimental.pallas import tpu_sc as plsc

SEQ_LEN = 16384
_INFO = plsc.get_sparse_core_info()
NC = _INFO.num_cores
NS = _INFO.num_subcores
NW = NC * NS
B_PER_W = SEQ_LEN // NW
IDX_CHUNK = 128
N_CHUNKS = B_PER_W // IDX_CHUNK
LANES = _INFO.num_lanes


def _sc_body(idx_hbm, kmer_hbm, site_hbm, out_hbm, idx_v, rows_v, site_v, sem):
    wid = lax.axis_index("s") * NC + lax.axis_index("c")
    base = wid * B_PER_W
    pltpu.sync_copy(idx_hbm.at[wid], idx_v)
    gathers = [
        pltpu.async_copy(
            kmer_hbm.at[idx_v.at[j]],
            rows_v.at[pl.ds(j * IDX_CHUNK, IDX_CHUNK)],
            sem,
        )
        for j in range(N_CHUNKS)
    ]
    pltpu.sync_copy(site_hbm.at[pl.ds(base, B_PER_W)], site_v)
    for g in gathers:
        g.wait()
    for i in range(B_PER_W // LANES):
        sl = pl.ds(i * LANES, LANES)
        rows_v[sl] = rows_v[sl] + site_v[sl]
    pltpu.sync_copy(rows_v, out_hbm.at[pl.ds(base, B_PER_W)])


@functools.partial(
    pl.kernel,
    out_type=jax.ShapeDtypeStruct((SEQ_LEN,), jnp.float32),
    mesh=plsc.VectorSubcoreMesh(core_axis_name="c", subcore_axis_name="s"),
    scratch_types=[
        pltpu.VMEM((N_CHUNKS, IDX_CHUNK), jnp.int32),
        pltpu.VMEM((B_PER_W,), jnp.float32),
        pltpu.VMEM((B_PER_W,), jnp.float32),
        pltpu.SemaphoreType.DMA,
    ],
)
def _sc_lookup(idx_hbm, kmer_hbm, site_hbm, out_hbm, idx_v, rows_v, site_v, sem):
    _sc_body(idx_hbm, kmer_hbm, site_hbm, out_hbm, idx_v, rows_v, site_v, sem)


@jax.jit
def kernel(encoded_parent, kmer_rates_weight, site_rates_weight):
    idx = encoded_parent.astype(jnp.int32).reshape(NW, N_CHUNKS, IDX_CHUNK)
    kmer = kmer_rates_weight.reshape(-1)
    site = site_rates_weight.reshape(-1)[:SEQ_LEN]
    return _sc_lookup(idx, kmer, site)


# speedup vs baseline: 1.4292x; 1.0067x over previous
import functools

import jax
import jax.numpy as jnp
from jax import lax
from jax.experimental import pallas as pl
from jax.experimental.pallas import tpu as pltpu
from jax.experimental.pallas import tpu_sc as plsc

SEQ_LEN = 16384
_INFO = plsc.get_sparse_core_info()
NC = _INFO.num_cores
NS = _INFO.num_subcores
NW = NC * NS
B_PER_W = SEQ_LEN // NW
IDX_CHUNK = 128
N_CHUNKS = B_PER_W // IDX_CHUNK
LANES = _INFO.num_lanes


def _sc_body(idx_hbm, kmer_hbm, site_hbm, out_hbm, idx_v, rows_v, site_v, sem):
    wid = lax.axis_index("s") * NC + lax.axis_index("c")
    base = wid * B_PER_W
    pltpu.sync_copy(idx_hbm.at[wid], idx_v)
    g = pltpu.async_copy(kmer_hbm.at[idx_v], rows_v, sem)
    pltpu.sync_copy(site_hbm.at[pl.ds(base, B_PER_W)], site_v)
    g.wait()

    def _add(i, carry):
        sl = pl.ds(i * LANES, LANES)
        rows_v[sl] = rows_v[sl] + site_v[sl]
        return carry

    lax.fori_loop(0, B_PER_W // LANES, _add, 0, unroll=4)
    pltpu.sync_copy(rows_v, out_hbm.at[pl.ds(base, B_PER_W)])


@functools.partial(
    pl.kernel,
    out_type=jax.ShapeDtypeStruct((SEQ_LEN,), jnp.float32),
    mesh=plsc.VectorSubcoreMesh(core_axis_name="c", subcore_axis_name="s"),
    scratch_types=[
        pltpu.VMEM((B_PER_W,), jnp.int32),
        pltpu.VMEM((B_PER_W,), jnp.float32),
        pltpu.VMEM((B_PER_W,), jnp.float32),
        pltpu.SemaphoreType.DMA,
    ],
)
def _sc_lookup(idx_hbm, kmer_hbm, site_hbm, out_hbm, idx_v, rows_v, site_v, sem):
    _sc_body(idx_hbm, kmer_hbm, site_hbm, out_hbm, idx_v, rows_v, site_v, sem)


@jax.jit
def kernel(encoded_parent, kmer_rates_weight, site_rates_weight):
    idx = encoded_parent.astype(jnp.int32).reshape(NW, B_PER_W)
    kmer = kmer_rates_weight.reshape(-1)
    site = site_rates_weight.reshape(-1)[:SEQ_LEN]
    return _sc_lookup(idx, kmer, site)


# speedup vs baseline: 1.4525x; 1.0163x over previous
import functools

import jax
import jax.numpy as jnp
from jax import lax
from jax.experimental import pallas as pl
from jax.experimental.pallas import tpu as pltpu
from jax.experimental.pallas import tpu_sc as plsc

SEQ_LEN = 16384
_INFO = plsc.get_sparse_core_info()
NC = _INFO.num_cores
NS = _INFO.num_subcores
NW = NC * NS
B_PER_W = SEQ_LEN // NW
HALF = B_PER_W // 2
LANES = _INFO.num_lanes


def _sc_body(idx_hbm, kmer_hbm, site_hbm, out_hbm,
             idx0_v, idx1_v, rows_v, site_v, sem_i0, sem_i1, sem_s, sem_g0, sem_g1):
    wid = lax.axis_index("s") * NC + lax.axis_index("c")
    base = wid * B_PER_W
    i0 = pltpu.async_copy(idx_hbm.at[wid, 0], idx0_v, sem_i0)
    i1 = pltpu.async_copy(idx_hbm.at[wid, 1], idx1_v, sem_i1)
    s = pltpu.async_copy(site_hbm.at[pl.ds(base, B_PER_W)], site_v, sem_s)
    i0.wait()
    g0 = pltpu.async_copy(kmer_hbm.at[idx0_v], rows_v.at[pl.ds(0, HALF)], sem_g0)
    i1.wait()
    g1 = pltpu.async_copy(kmer_hbm.at[idx1_v], rows_v.at[pl.ds(HALF, HALF)], sem_g1)
    s.wait()

    def _add(i, carry):
        sl = pl.ds(i * LANES, LANES)
        rows_v[sl] = rows_v[sl] + site_v[sl]
        return carry

    g0.wait()
    lax.fori_loop(0, HALF // LANES, _add, 0, unroll=4)
    g1.wait()
    lax.fori_loop(HALF // LANES, B_PER_W // LANES, _add, 0, unroll=4)
    pltpu.sync_copy(rows_v, out_hbm.at[pl.ds(base, B_PER_W)])


@functools.partial(
    pl.kernel,
    out_type=jax.ShapeDtypeStruct((SEQ_LEN,), jnp.float32),
    mesh=plsc.VectorSubcoreMesh(core_axis_name="c", subcore_axis_name="s"),
    scratch_types=[
        pltpu.VMEM((HALF,), jnp.int32),
        pltpu.VMEM((HALF,), jnp.int32),
        pltpu.VMEM((B_PER_W,), jnp.float32),
        pltpu.VMEM((B_PER_W,), jnp.float32),
        pltpu.SemaphoreType.DMA,
        pltpu.SemaphoreType.DMA,
        pltpu.SemaphoreType.DMA,
        pltpu.SemaphoreType.DMA,
        pltpu.SemaphoreType.DMA,
    ],
)
def _sc_lookup(idx_hbm, kmer_hbm, site_hbm, out_hbm,
               idx0_v, idx1_v, rows_v, site_v, sem_i0, sem_i1, sem_s, sem_g0, sem_g1):
    _sc_body(idx_hbm, kmer_hbm, site_hbm, out_hbm,
             idx0_v, idx1_v, rows_v, site_v, sem_i0, sem_i1, sem_s, sem_g0, sem_g1)


@jax.jit
def kernel(encoded_parent, kmer_rates_weight, site_rates_weight):
    idx = encoded_parent.astype(jnp.int32).reshape(NW, 2, HALF)
    kmer = kmer_rates_weight.reshape(-1)
    site = site_rates_weight.reshape(-1)[:SEQ_LEN]
    return _sc_lookup(idx, kmer, site)
